# Initial kernel scaffold; baseline (speedup 1.0000x reference)
import jax
import jax.numpy as jnp
from jax.experimental import pallas as pl


def kernel(inputs, indices, features, labels):
    raise NotImplementedError("write your pallas kernel here")


# speedup vs baseline: 2.7133x; 2.7133x over previous
import functools

import jax
import jax.numpy as jnp
from jax import lax
from jax.experimental import pallas as pl
from jax.experimental.pallas import tpu as pltpu
from jax.experimental.pallas import tpu_sc as plsc

_BATCH = 1024
_N = 100000
_F = 32
_INV_TEMP = 20.0
_SHIFT = 20.0

_BLK = 2048
_NBLK = (_N + _BLK - 1) // _BLK

_NC = 2
_NS = 16
_NW = _NC * _NS
_BPW = _BATCH // _NW
_REG = 16


def _sc_gather128(idx, table, shift, out_dtype):
    mesh = plsc.VectorSubcoreMesh(core_axis_name="c", subcore_axis_name="s")

    @functools.partial(
        pl.kernel,
        mesh=mesh,
        out_type=jax.ShapeDtypeStruct((_BATCH, 128), out_dtype),
        scratch_types=[
            pltpu.VMEM((_BPW,), jnp.int32),
            pltpu.VMEM((_BPW,), jnp.int32),
            pltpu.VMEM((_BPW, 128), out_dtype),
            pltpu.SemaphoreType.DMA,
        ],
    )
    def k(idx_hbm, tab_hbm, out_hbm, idx_v, q_v, rows_v, sem):
        wid = lax.axis_index("s") * _NC + lax.axis_index("c")
        base = wid * _BPW
        pltpu.sync_copy(idx_hbm.at[pl.ds(base, _BPW)], idx_v)

        @pl.loop(0, _BPW, step=_REG)
        def _(j):
            sl = pl.ds(j, _REG)
            q_v.at[sl][...] = lax.shift_right_logical(idx_v.at[sl][...], shift)

        pltpu.async_copy(tab_hbm.at[q_v], rows_v, sem).wait()
        pltpu.sync_copy(rows_v, out_hbm.at[pl.ds(base, _BPW)])

    return k(idx, table)


def _tc_extract_body(lab_ref, idx_ref, t_ref):
    r = idx_ref[...] & 127
    col = lax.broadcasted_iota(jnp.int32, (_BATCH, 128), 1)
    t_ref[...] = jnp.sum(jnp.where(col == r, lab_ref[...], 0), axis=1,
                         keepdims=True)


def _tc_extract(labrows, idx2d):
    return pl.pallas_call(
        _tc_extract_body,
        out_shape=jax.ShapeDtypeStruct((_BATCH, 1), jnp.int32),
    )(labrows, idx2d)


def _tc_sumexp_body(inp_ref, feat_ref, acc_ref):
    i = pl.program_id(0)

    @pl.when(i == 0)
    def _():
        acc_ref[...] = jnp.zeros_like(acc_ref)

    logits = lax.dot_general(
        inp_ref[...], feat_ref[...],
        dimension_numbers=(((1,), (1,)), ((), ())),
        preferred_element_type=jnp.float32,
    )
    e = jnp.exp(logits * _INV_TEMP - _SHIFT)
    col = i * _BLK + lax.broadcasted_iota(jnp.int32, (_BATCH, _BLK), 1)
    e = jnp.where(col < _N, e, 0.0)
    acc_ref[...] += jnp.sum(e, axis=1, keepdims=True)


def _tc_sumexp(inputs, features):
    return pl.pallas_call(
        _tc_sumexp_body,
        grid=(_NBLK,),
        in_specs=[
            pl.BlockSpec((_BATCH, _F), lambda i: (0, 0)),
            pl.BlockSpec((_BLK, _F), lambda i: (i, 0)),
        ],
        out_specs=pl.BlockSpec((_BATCH, 1), lambda i: (0, 0)),
        out_shape=jax.ShapeDtypeStruct((_BATCH, 1), jnp.float32),
    )(inputs, features)


def _tc_finish_body(inp_ref, rows_ref, t_ref, se_ref, out_ref):
    rem = t_ref[...] & 3
    picked = jnp.zeros((_BATCH, 1), jnp.float32)
    for k in range(4):
        dk = jnp.sum(inp_ref[...] * rows_ref[:, 32 * k:32 * (k + 1)],
                     axis=1, keepdims=True)
        picked = picked + jnp.where(rem == k, dk, 0.0)
    lse = jnp.log(se_ref[...]) + _SHIFT
    out_ref[...] = jnp.mean(lse - picked * _INV_TEMP)[None, None]


def _tc_finish(inputs, featrows, t, sumexp):
    return pl.pallas_call(
        _tc_finish_body,
        out_shape=jax.ShapeDtypeStruct((1, 1), jnp.float32),
    )(inputs, featrows, t, sumexp)


def kernel(inputs, indices, features, labels):
    labpad = jnp.pad(labels, (0, 782 * 128 - _N)).reshape(782, 128)
    feat4 = features.reshape(_N // 4, 128)
    labrows = _sc_gather128(indices, labpad, 7, jnp.int32)
    t = _tc_extract(labrows, indices.reshape(_BATCH, 1))
    featrows = _sc_gather128(t.reshape(_BATCH), feat4, 2, jnp.float32)
    sumexp = _tc_sumexp(inputs, features)
    loss = _tc_finish(inputs, featrows, t, sumexp)
    return loss[0, 0]


# speedup vs baseline: 3.4032x; 1.2543x over previous
import functools

import jax
import jax.numpy as jnp
from jax import lax
from jax.experimental import pallas as pl
from jax.experimental.pallas import tpu as pltpu
from jax.experimental.pallas import tpu_sc as plsc

_BATCH = 1024
_N = 100000
_F = 32
_INV_TEMP = 20.0
_SHIFT = 20.0

_BLK = 4096
_NBLK = 24
_TBLK = 2048
_TIDX = 48

_NC = 2
_NS = 16
_NW = _NC * _NS
_BPW = _BATCH // _NW
_REG = 16


def _sc_gather128(idx, table, shift, out_dtype):
    mesh = plsc.VectorSubcoreMesh(core_axis_name="c", subcore_axis_name="s")

    @functools.partial(
        pl.kernel,
        mesh=mesh,
        out_type=jax.ShapeDtypeStruct((_BATCH, 128), out_dtype),
        scratch_types=[
            pltpu.VMEM((_BPW,), jnp.int32),
            pltpu.VMEM((_BPW,), jnp.int32),
            pltpu.VMEM((_BPW, 128), out_dtype),
            pltpu.SemaphoreType.DMA,
        ],
    )
    def k(idx_hbm, tab_hbm, out_hbm, idx_v, q_v, rows_v, sem):
        wid = lax.axis_index("s") * _NC + lax.axis_index("c")
        base = wid * _BPW
        pltpu.sync_copy(idx_hbm.at[pl.ds(base, _BPW)], idx_v)

        @pl.loop(0, _BPW, step=_REG)
        def _(j):
            sl = pl.ds(j, _REG)
            q_v.at[sl][...] = lax.shift_right_logical(idx_v.at[sl][...], shift)

        pltpu.async_copy(tab_hbm.at[q_v], rows_v, sem).wait()
        pltpu.sync_copy(rows_v, out_hbm.at[pl.ds(base, _BPW)])

    return k(idx, table)


def _tc_extract_body(lab_ref, idx_ref, t_ref):
    r = idx_ref[...] & 127
    col = lax.broadcasted_iota(jnp.int32, (_BATCH, 128), 1)
    t_ref[...] = jnp.sum(jnp.where(col == r, lab_ref[...], 0), axis=1,
                         keepdims=True)


def _tc_extract(labrows, idx2d):
    return pl.pallas_call(
        _tc_extract_body,
        out_shape=jax.ShapeDtypeStruct((_BATCH, 1), jnp.int32),
    )(labrows, idx2d)


def _lane_chunk_sum(e, width):
    s = e[:, 0:128]
    for c in range(1, width // 128):
        s = s + e[:, 128 * c:128 * (c + 1)]
    return s


def _tc_sumexp_body(inp_ref, feat_ref, acc_ref):
    i = pl.program_id(0)
    inp = (inp_ref[...] * _INV_TEMP).astype(jnp.bfloat16)
    logits = lax.dot_general(
        inp, feat_ref[...].astype(jnp.bfloat16),
        dimension_numbers=(((1,), (1,)), ((), ())),
        preferred_element_type=jnp.float32,
    )
    s = _lane_chunk_sum(jnp.exp(logits - _SHIFT), _BLK)

    @pl.when(i == 0)
    def _():
        acc_ref[...] = s

    @pl.when(i > 0)
    def _():
        acc_ref[...] += s


def _tc_sumexp(inputs, features):
    return pl.pallas_call(
        _tc_sumexp_body,
        grid=(_NBLK,),
        in_specs=[
            pl.BlockSpec((_BATCH, _F), lambda i: (0, 0)),
            pl.BlockSpec((_BLK, _F), lambda i: (i, 0)),
        ],
        out_specs=pl.BlockSpec((_BATCH, 128), lambda i: (0, 0)),
        out_shape=jax.ShapeDtypeStruct((_BATCH, 128), jnp.float32),
    )(inputs, features)


def _tc_finish_body(inp_ref, feat_ref, rows_ref, t_ref, acc_ref, out_ref):
    inp = (inp_ref[...] * _INV_TEMP).astype(jnp.bfloat16)
    logits = lax.dot_general(
        inp, feat_ref[...].astype(jnp.bfloat16),
        dimension_numbers=(((1,), (1,)), ((), ())),
        preferred_element_type=jnp.float32,
    )
    col = _TIDX * _TBLK + lax.broadcasted_iota(jnp.int32, (_BATCH, _TBLK), 1)
    e = jnp.where(col < _N, jnp.exp(logits - _SHIFT), 0.0)
    se = jnp.sum(acc_ref[...] + _lane_chunk_sum(e, _TBLK), axis=1,
                 keepdims=True)
    lse = jnp.log(se) + _SHIFT

    rem = t_ref[...] & 3
    picked = jnp.zeros((_BATCH, 1), jnp.float32)
    for k in range(4):
        dk = jnp.sum(inp_ref[...] * rows_ref[:, 32 * k:32 * (k + 1)],
                     axis=1, keepdims=True)
        picked = picked + jnp.where(rem == k, dk, 0.0)
    out_ref[...] = jnp.mean(lse - picked * _INV_TEMP)[None, None]


def _tc_finish(inputs, features, featrows, t, acc):
    return pl.pallas_call(
        _tc_finish_body,
        grid=(1,),
        in_specs=[
            pl.BlockSpec((_BATCH, _F), lambda i: (0, 0)),
            pl.BlockSpec((_TBLK, _F), lambda i: (_TIDX, 0)),
            pl.BlockSpec((_BATCH, 128), lambda i: (0, 0)),
            pl.BlockSpec((_BATCH, 1), lambda i: (0, 0)),
            pl.BlockSpec((_BATCH, 128), lambda i: (0, 0)),
        ],
        out_specs=pl.BlockSpec((1, 1), lambda i: (0, 0)),
        out_shape=jax.ShapeDtypeStruct((1, 1), jnp.float32),
    )(inputs, features, featrows, t, acc)


def kernel(inputs, indices, features, labels):
    labpad = jnp.pad(labels, (0, 782 * 128 - _N)).reshape(782, 128)
    feat4 = features.reshape(_N // 4, 128)
    labrows = _sc_gather128(indices, labpad, 7, jnp.int32)
    t = _tc_extract(labrows, indices.reshape(_BATCH, 1))
    featrows = _sc_gather128(t.reshape(_BATCH), feat4, 2, jnp.float32)
    acc = _tc_sumexp(inputs, features)
    loss = _tc_finish(inputs, features, featrows, t, acc)
    return loss[0, 0]


# speedup vs baseline: 3.8477x; 1.1306x over previous
import functools

import jax
import jax.numpy as jnp
from jax import lax
from jax.experimental import pallas as pl
from jax.experimental.pallas import tpu as pltpu
from jax.experimental.pallas import tpu_sc as plsc

_BATCH = 1024
_N = 100000
_F = 32
_INV_TEMP = 20.0
_SHIFT = 20.0

_BLK = 4096
_NBLK = 24
_TBLK = 2048
_TIDX = 48

_NC = 2
_NS = 16
_NW = _NC * _NS
_BPW = _BATCH // _NW
_REG = 16


def _sc_gather128(idx, table, shift, out_dtype):
    mesh = plsc.VectorSubcoreMesh(core_axis_name="c", subcore_axis_name="s")

    @functools.partial(
        pl.kernel,
        mesh=mesh,
        out_type=jax.ShapeDtypeStruct((_BATCH, 128), out_dtype),
        scratch_types=[
            pltpu.VMEM((_BPW,), jnp.int32),
            pltpu.VMEM((_BPW,), jnp.int32),
            pltpu.VMEM((_BPW, 128), out_dtype),
            pltpu.SemaphoreType.DMA,
        ],
    )
    def k(idx_hbm, tab_hbm, out_hbm, idx_v, q_v, rows_v, sem):
        wid = lax.axis_index("s") * _NC + lax.axis_index("c")
        base = wid * _BPW
        pltpu.sync_copy(idx_hbm.at[pl.ds(base, _BPW)], idx_v)

        @pl.loop(0, _BPW, step=_REG)
        def _(j):
            sl = pl.ds(j, _REG)
            q_v.at[sl][...] = lax.shift_right_logical(idx_v.at[sl][...], shift)

        pltpu.async_copy(tab_hbm.at[q_v], rows_v, sem).wait()
        pltpu.sync_copy(rows_v, out_hbm.at[pl.ds(base, _BPW)])

    return k(idx, table)


def _tc_extract_body(lab_ref, idx_ref, t_ref):
    r = idx_ref[...] & 127
    col = lax.broadcasted_iota(jnp.int32, (_BATCH, 128), 1)
    t_ref[...] = jnp.sum(jnp.where(col == r, lab_ref[...], 0), axis=1,
                         keepdims=True)


def _tc_extract(labrows, idx2d):
    return pl.pallas_call(
        _tc_extract_body,
        out_shape=jax.ShapeDtypeStruct((_BATCH, 1), jnp.int32),
    )(labrows, idx2d)


def _lane_chunk_sum(e, width):
    s = e[:, 0:128]
    for c in range(1, width // 128):
        s = s + e[:, 128 * c:128 * (c + 1)]
    return s


def _tc_sumexp_body(inp_ref, feat_ref, acc_ref):
    i = pl.program_id(0)
    inp = (inp_ref[...] * _INV_TEMP).astype(jnp.bfloat16)
    logits = lax.dot_general(
        inp, feat_ref[...].astype(jnp.bfloat16),
        dimension_numbers=(((1,), (0,)), ((), ())),
        preferred_element_type=jnp.float32,
    )
    s = _lane_chunk_sum(jnp.exp(logits - _SHIFT), _BLK)

    @pl.when(i == 0)
    def _():
        acc_ref[...] = s

    @pl.when(i > 0)
    def _():
        acc_ref[...] += s


def _tc_sumexp(inputs, featT):
    return pl.pallas_call(
        _tc_sumexp_body,
        grid=(_NBLK,),
        in_specs=[
            pl.BlockSpec((_BATCH, _F), lambda i: (0, 0)),
            pl.BlockSpec((_F, _BLK), lambda i: (0, i)),
        ],
        out_specs=pl.BlockSpec((_BATCH, 128), lambda i: (0, 0)),
        out_shape=jax.ShapeDtypeStruct((_BATCH, 128), jnp.float32),
    )(inputs, featT)


def _tc_finish_body(inp_ref, feat_ref, rows_ref, t_ref, acc_ref, out_ref):
    inp = (inp_ref[...] * _INV_TEMP).astype(jnp.bfloat16)
    logits = lax.dot_general(
        inp, feat_ref[...].astype(jnp.bfloat16),
        dimension_numbers=(((1,), (0,)), ((), ())),
        preferred_element_type=jnp.float32,
    )
    col = _TIDX * _TBLK + lax.broadcasted_iota(jnp.int32, (_BATCH, _TBLK), 1)
    e = jnp.where(col < _N, jnp.exp(logits - _SHIFT), 0.0)
    se = jnp.sum(acc_ref[...] + _lane_chunk_sum(e, _TBLK), axis=1,
                 keepdims=True)
    lse = jnp.log(se) + _SHIFT

    rem = t_ref[...] & 3
    picked = jnp.zeros((_BATCH, 1), jnp.float32)
    for k in range(4):
        dk = jnp.sum(inp_ref[...] * rows_ref[:, 32 * k:32 * (k + 1)],
                     axis=1, keepdims=True)
        picked = picked + jnp.where(rem == k, dk, 0.0)
    out_ref[...] = jnp.mean(lse - picked * _INV_TEMP)[None, None]


def _tc_finish(inputs, featT, featrows, t, acc):
    return pl.pallas_call(
        _tc_finish_body,
        grid=(1,),
        in_specs=[
            pl.BlockSpec((_BATCH, _F), lambda i: (0, 0)),
            pl.BlockSpec((_F, _TBLK), lambda i: (0, _TIDX)),
            pl.BlockSpec((_BATCH, 128), lambda i: (0, 0)),
            pl.BlockSpec((_BATCH, 1), lambda i: (0, 0)),
            pl.BlockSpec((_BATCH, 128), lambda i: (0, 0)),
        ],
        out_specs=pl.BlockSpec((1, 1), lambda i: (0, 0)),
        out_shape=jax.ShapeDtypeStruct((1, 1), jnp.float32),
    )(inputs, featT, featrows, t, acc)


def kernel(inputs, indices, features, labels):
    labpad = jnp.pad(labels, (0, 782 * 128 - _N)).reshape(782, 128)
    featT = features.T
    feat4 = features.reshape(_N // 4, 128)
    labrows = _sc_gather128(indices, labpad, 7, jnp.int32)
    t = _tc_extract(labrows, indices.reshape(_BATCH, 1))
    featrows = _sc_gather128(t.reshape(_BATCH), feat4, 2, jnp.float32)
    acc = _tc_sumexp(inputs, featT)
    loss = _tc_finish(inputs, featT, featrows, t, acc)
    return loss[0, 0]


# speedup vs baseline: 4.1338x; 1.0743x over previous
import functools

import jax
import jax.numpy as jnp
from jax import lax
from jax.experimental import pallas as pl
from jax.experimental.pallas import tpu as pltpu
from jax.experimental.pallas import tpu_sc as plsc

_BATCH = 1024
_N = 100000
_F = 32
_INV_TEMP = 20.0
_SHIFT = 20.0

_BLK = 4096
_NBLK = 24
_TBLK = 2048
_TIDX = 48

_NC = 2
_NS = 16
_NW = _NC * _NS
_BPW = _BATCH // _NW
_REG = 16


def _sc_gather128(idx, table, shift, out_dtype):
    mesh = plsc.VectorSubcoreMesh(core_axis_name="c", subcore_axis_name="s")

    @functools.partial(
        pl.kernel,
        mesh=mesh,
        out_type=jax.ShapeDtypeStruct((_BATCH, 128), out_dtype),
        scratch_types=[
            pltpu.VMEM((_BPW,), jnp.int32),
            pltpu.VMEM((_BPW,), jnp.int32),
            pltpu.VMEM((_BPW, 128), out_dtype),
            pltpu.SemaphoreType.DMA,
        ],
    )
    def k(idx_hbm, tab_hbm, out_hbm, idx_v, q_v, rows_v, sem):
        wid = lax.axis_index("s") * _NC + lax.axis_index("c")
        base = wid * _BPW
        pltpu.sync_copy(idx_hbm.at[pl.ds(base, _BPW)], idx_v)

        @pl.loop(0, _BPW, step=_REG)
        def _(j):
            sl = pl.ds(j, _REG)
            q_v.at[sl][...] = lax.shift_right_logical(idx_v.at[sl][...], shift)

        pltpu.async_copy(tab_hbm.at[q_v], rows_v, sem).wait()
        pltpu.sync_copy(rows_v, out_hbm.at[pl.ds(base, _BPW)])

    return k(idx, table)


def _tc_extract_body(lab_ref, idx_ref, t_ref):
    r = idx_ref[...] & 127
    col = lax.broadcasted_iota(jnp.int32, (_BATCH, 128), 1)
    t_ref[...] = jnp.sum(jnp.where(col == r, lab_ref[...], 0), axis=1,
                         keepdims=True)


def _tc_extract(labrows, idx2d):
    return pl.pallas_call(
        _tc_extract_body,
        out_shape=jax.ShapeDtypeStruct((_BATCH, 1), jnp.int32),
    )(labrows, idx2d)


def _lane_chunk_sum(e, width):
    s = e[:, 0:128]
    for c in range(1, width // 128):
        s = s + e[:, 128 * c:128 * (c + 1)]
    return s


def _tc_sumexp_body(inp_ref, feat_ref, acc_ref):
    i = pl.program_id(0)
    inp = (inp_ref[...] * _INV_TEMP).astype(jnp.bfloat16)
    logits = lax.dot_general(
        inp, feat_ref[...].astype(jnp.bfloat16),
        dimension_numbers=(((1,), (0,)), ((), ())),
        preferred_element_type=jnp.float32,
    )
    s = _lane_chunk_sum(jnp.exp(logits - _SHIFT), _BLK)

    @pl.when(i == 0)
    def _():
        acc_ref[...] = s

    @pl.when(i > 0)
    def _():
        acc_ref[...] += s


def _tc_sumexp(inputs, featT):
    return pl.pallas_call(
        _tc_sumexp_body,
        grid=(_NBLK,),
        in_specs=[
            pl.BlockSpec((_BATCH, _F), lambda i: (0, 0)),
            pl.BlockSpec((_F, _BLK), lambda i: (0, i)),
        ],
        out_specs=pl.BlockSpec((_BATCH, 128), lambda i: (0, 0)),
        out_shape=jax.ShapeDtypeStruct((_BATCH, 128), jnp.float32),
    )(inputs, featT)


def _tc_finish_body(inp_ref, feat_ref, rows_ref, t_ref, acc_ref, out_ref):
    inp = (inp_ref[...] * _INV_TEMP).astype(jnp.bfloat16)
    logits = lax.dot_general(
        inp, feat_ref[...].astype(jnp.bfloat16),
        dimension_numbers=(((1,), (0,)), ((), ())),
        preferred_element_type=jnp.float32,
    )
    col = _TIDX * _TBLK + lax.broadcasted_iota(jnp.int32, (_BATCH, _TBLK), 1)
    e = jnp.where(col < _N, jnp.exp(logits - _SHIFT), 0.0)
    se = jnp.sum(acc_ref[...] + _lane_chunk_sum(e, _TBLK), axis=1,
                 keepdims=True)
    lse = jnp.log(se) + _SHIFT

    rem = t_ref[...] & 3
    picked = jnp.zeros((_BATCH, 1), jnp.float32)
    for k in range(4):
        dk = jnp.sum(inp_ref[...] * rows_ref[:, 32 * k:32 * (k + 1)],
                     axis=1, keepdims=True)
        picked = picked + jnp.where(rem == k, dk, 0.0)
    out_ref[...] = jnp.mean(lse - picked * _INV_TEMP)[None, None]


def _tc_finish(inputs, featT, featrows, t, acc):
    return pl.pallas_call(
        _tc_finish_body,
        grid=(1,),
        in_specs=[
            pl.BlockSpec((_BATCH, _F), lambda i: (0, 0)),
            pl.BlockSpec((_F, _TBLK), lambda i: (0, _TIDX)),
            pl.BlockSpec((_BATCH, 128), lambda i: (0, 0)),
            pl.BlockSpec((_BATCH, 1), lambda i: (0, 0)),
            pl.BlockSpec((_BATCH, 128), lambda i: (0, 0)),
        ],
        out_specs=pl.BlockSpec((1, 1), lambda i: (0, 0)),
        out_shape=jax.ShapeDtypeStruct((1, 1), jnp.float32),
    )(inputs, featT, featrows, t, acc)


def kernel(inputs, indices, features, labels):
    labpad = jnp.pad(labels, (0, 782 * 128 - _N)).reshape(782, 128)
    featT = features.T
    feat4 = features.reshape(_N // 4, 128)
    labrows = _sc_gather128(indices, labpad, 7, jnp.int32)
    t = _tc_extract(labrows, indices.reshape(_BATCH, 1))
    acc = _tc_sumexp(inputs, featT)
    t_dep = t + (acc[0:1, 0:1] * 0.0).astype(jnp.int32)
    featrows = _sc_gather128(t_dep.reshape(_BATCH), feat4, 2, jnp.float32)
    loss = _tc_finish(inputs, featT, featrows, t, acc)
    return loss[0, 0]


# speedup vs baseline: 4.3614x; 1.0551x over previous
import functools

import jax
import jax.numpy as jnp
from jax import lax
from jax.experimental import pallas as pl
from jax.experimental.pallas import tpu as pltpu
from jax.experimental.pallas import tpu_sc as plsc

_BATCH = 1024
_N = 100000
_F = 32
_INV_TEMP = 20.0
_SHIFT = 20.0

_BLK = 4096
_NBLK = 24
_TBLK = 2048
_TIDX = 48

_NC = 2
_NS = 16
_NW = _NC * _NS
_BPW = _BATCH // _NW
_REG = 16


def _sc_gather128(idx, table, shift, out_dtype):
    mesh = plsc.VectorSubcoreMesh(core_axis_name="c", subcore_axis_name="s")

    @functools.partial(
        pl.kernel,
        mesh=mesh,
        out_type=jax.ShapeDtypeStruct((_BATCH, 128), out_dtype),
        scratch_types=[
            pltpu.VMEM((_BPW,), jnp.int32),
            pltpu.VMEM((_BPW,), jnp.int32),
            pltpu.VMEM((_BPW, 128), out_dtype),
            pltpu.SemaphoreType.DMA,
        ],
    )
    def k(idx_hbm, tab_hbm, out_hbm, idx_v, q_v, rows_v, sem):
        wid = lax.axis_index("s") * _NC + lax.axis_index("c")
        base = wid * _BPW
        pltpu.sync_copy(idx_hbm.at[pl.ds(base, _BPW)], idx_v)

        @pl.loop(0, _BPW, step=_REG)
        def _(j):
            sl = pl.ds(j, _REG)
            q_v.at[sl][...] = lax.shift_right_logical(idx_v.at[sl][...], shift)

        pltpu.async_copy(tab_hbm.at[q_v], rows_v, sem).wait()
        pltpu.sync_copy(rows_v, out_hbm.at[pl.ds(base, _BPW)])

    return k(idx, table)


def _tc_extract_body(lab_ref, idx_ref, t_ref):
    r = idx_ref[...] & 127
    col = lax.broadcasted_iota(jnp.int32, (_BATCH, 128), 1)
    t_ref[...] = jnp.sum(jnp.where(col == r, lab_ref[...], 0), axis=1,
                         keepdims=True)


def _tc_extract(labrows, idx2d):
    return pl.pallas_call(
        _tc_extract_body,
        out_shape=jax.ShapeDtypeStruct((_BATCH, 1), jnp.int32),
    )(labrows, idx2d)


def _lane_chunk_sum(e, width):
    s = e[:, 0:128]
    for c in range(1, width // 128):
        s = s + e[:, 128 * c:128 * (c + 1)]
    return s


def _pick_chunks(logits, p, width):
    pcb = lax.shift_right_arithmetic(p, 7) + jnp.zeros((_BATCH, 128),
                                                       jnp.int32)
    pk = jnp.where(pcb == 0, logits[:, 0:128], 0.0)
    for c in range(1, width // 128):
        pk = jnp.where(pcb == c, logits[:, 128 * c:128 * (c + 1)], pk)
    return pk


def _tc_main_body(inp_ref, feat_ref, t_ref, acc_ref, pick_ref):
    i = pl.program_id(0)
    inp = (inp_ref[...] * _INV_TEMP).astype(jnp.bfloat16)
    logits = lax.dot_general(
        inp, feat_ref[...].astype(jnp.bfloat16),
        dimension_numbers=(((1,), (0,)), ((), ())),
        preferred_element_type=jnp.float32,
    )
    s = _lane_chunk_sum(jnp.exp(logits - _SHIFT), _BLK)
    pk = _pick_chunks(logits, t_ref[...] - i * _BLK, _BLK)

    @pl.when(i == 0)
    def _():
        acc_ref[...] = s
        pick_ref[...] = pk

    @pl.when(i > 0)
    def _():
        acc_ref[...] += s
        pick_ref[...] += pk


def _tc_main(inputs, featT, t):
    return pl.pallas_call(
        _tc_main_body,
        grid=(_NBLK,),
        in_specs=[
            pl.BlockSpec((_BATCH, _F), lambda i: (0, 0)),
            pl.BlockSpec((_F, _BLK), lambda i: (0, i)),
            pl.BlockSpec((_BATCH, 1), lambda i: (0, 0)),
        ],
        out_specs=[
            pl.BlockSpec((_BATCH, 128), lambda i: (0, 0)),
            pl.BlockSpec((_BATCH, 128), lambda i: (0, 0)),
        ],
        out_shape=[
            jax.ShapeDtypeStruct((_BATCH, 128), jnp.float32),
            jax.ShapeDtypeStruct((_BATCH, 128), jnp.float32),
        ],
    )(inputs, featT, t)


def _tc_finish_body(inp_ref, feat_ref, t_ref, acc_ref, pick_ref, out_ref):
    t = t_ref[...]

    inp = (inp_ref[...] * _INV_TEMP).astype(jnp.bfloat16)
    logits = lax.dot_general(
        inp, feat_ref[...].astype(jnp.bfloat16),
        dimension_numbers=(((1,), (0,)), ((), ())),
        preferred_element_type=jnp.float32,
    )
    col = _TIDX * _TBLK + lax.broadcasted_iota(jnp.int32, (_BATCH, _TBLK), 1)
    e = jnp.where(col < _N, jnp.exp(logits - _SHIFT), 0.0)
    se = jnp.sum(acc_ref[...] + _lane_chunk_sum(e, _TBLK), axis=1,
                 keepdims=True)
    lse = jnp.log(se) + _SHIFT

    pk = pick_ref[...] + _pick_chunks(logits, t - _TIDX * _TBLK, _TBLK)
    lane = lax.broadcasted_iota(jnp.int32, (_BATCH, 128), 1)
    picked = jnp.sum(jnp.where(lane == (t & 127), pk, 0.0), axis=1,
                     keepdims=True)
    out_ref[...] = jnp.mean(lse - picked)[None, None]


def _tc_finish(inputs, featT, t, acc, pick):
    return pl.pallas_call(
        _tc_finish_body,
        grid=(1,),
        in_specs=[
            pl.BlockSpec((_BATCH, _F), lambda i: (0, 0)),
            pl.BlockSpec((_F, _TBLK), lambda i: (0, _TIDX)),
            pl.BlockSpec((_BATCH, 1), lambda i: (0, 0)),
            pl.BlockSpec((_BATCH, 128), lambda i: (0, 0)),
            pl.BlockSpec((_BATCH, 128), lambda i: (0, 0)),
        ],
        out_specs=pl.BlockSpec((1, 1), lambda i: (0, 0)),
        out_shape=jax.ShapeDtypeStruct((1, 1), jnp.float32),
    )(inputs, featT, t, acc, pick)


def kernel(inputs, indices, features, labels):
    labpad = jnp.pad(labels, (0, 782 * 128 - _N)).reshape(782, 128)
    featT = features.T
    labrows = _sc_gather128(indices, labpad, 7, jnp.int32)
    t = _tc_extract(labrows, indices.reshape(_BATCH, 1))
    acc, pick = _tc_main(inputs, featT, t)
    loss = _tc_finish(inputs, featT, t, acc, pick)
    return loss[0, 0]


# speedup vs baseline: 4.4967x; 1.0310x over previous
import functools

import jax
import jax.numpy as jnp
from jax import lax
from jax.experimental import pallas as pl
from jax.experimental.pallas import tpu as pltpu
from jax.experimental.pallas import tpu_sc as plsc

_BATCH = 1024
_N = 100000
_F = 32
_INV_TEMP = 20.0
_SHIFT = 20.0

_BLK = 8192
_NBLK = 12
_TBLK = 2048
_TIDX = 48

_NC = 2
_NS = 16
_NW = _NC * _NS
_BPW = _BATCH // _NW
_REG = 16


def _sc_gather128(idx, table, shift, out_dtype):
    mesh = plsc.VectorSubcoreMesh(core_axis_name="c", subcore_axis_name="s")

    @functools.partial(
        pl.kernel,
        mesh=mesh,
        out_type=jax.ShapeDtypeStruct((_BATCH, 128), out_dtype),
        scratch_types=[
            pltpu.VMEM((_BPW,), jnp.int32),
            pltpu.VMEM((_BPW,), jnp.int32),
            pltpu.VMEM((_BPW, 128), out_dtype),
            pltpu.SemaphoreType.DMA,
        ],
    )
    def k(idx_hbm, tab_hbm, out_hbm, idx_v, q_v, rows_v, sem):
        wid = lax.axis_index("s") * _NC + lax.axis_index("c")
        base = wid * _BPW
        pltpu.sync_copy(idx_hbm.at[pl.ds(base, _BPW)], idx_v)

        @pl.loop(0, _BPW, step=_REG)
        def _(j):
            sl = pl.ds(j, _REG)
            q_v.at[sl][...] = lax.shift_right_logical(idx_v.at[sl][...], shift)

        pltpu.async_copy(tab_hbm.at[q_v], rows_v, sem).wait()
        pltpu.sync_copy(rows_v, out_hbm.at[pl.ds(base, _BPW)])

    return k(idx, table)


def _extract_t(lab, idx):
    r = idx & 127
    col = lax.broadcasted_iota(jnp.int32, (_BATCH, 128), 1)
    return jnp.sum(jnp.where(col == r, lab, 0), axis=1, keepdims=True)


def _lane_chunk_sum(e, width):
    s = e[:, 0:128]
    for c in range(1, width // 128):
        s = s + e[:, 128 * c:128 * (c + 1)]
    return s


def _pick_chunks(logits, p, width):
    pcb = lax.shift_right_arithmetic(p, 7) + jnp.zeros((_BATCH, 128),
                                                       jnp.int32)
    pk = jnp.where(pcb == 0, logits[:, 0:128], 0.0)
    for c in range(1, width // 128):
        pk = jnp.where(pcb == c, logits[:, 128 * c:128 * (c + 1)], pk)
    return pk


def _tc_main_body(inp_ref, feat_ref, lab_ref, idx_ref, acc_ref, pick_ref,
                  t_ref):
    i = pl.program_id(0)

    @pl.when(i == 0)
    def _():
        t_ref[...] = _extract_t(lab_ref[...], idx_ref[...])

    inp = (inp_ref[...] * _INV_TEMP).astype(jnp.bfloat16)
    logits = lax.dot_general(
        inp, feat_ref[...].astype(jnp.bfloat16),
        dimension_numbers=(((1,), (0,)), ((), ())),
        preferred_element_type=jnp.float32,
    )
    s = _lane_chunk_sum(jnp.exp(logits - _SHIFT), _BLK)
    pk = _pick_chunks(logits, t_ref[...] - i * _BLK, _BLK)

    @pl.when(i == 0)
    def _():
        acc_ref[...] = s
        pick_ref[...] = pk

    @pl.when(i > 0)
    def _():
        acc_ref[...] += s
        pick_ref[...] += pk


def _tc_main(inputs, featT, labrows, idx2d):
    return pl.pallas_call(
        _tc_main_body,
        grid=(_NBLK,),
        in_specs=[
            pl.BlockSpec((_BATCH, _F), lambda i: (0, 0)),
            pl.BlockSpec((_F, _BLK), lambda i: (0, i)),
            pl.BlockSpec((_BATCH, 128), lambda i: (0, 0)),
            pl.BlockSpec((_BATCH, 1), lambda i: (0, 0)),
        ],
        out_specs=[
            pl.BlockSpec((_BATCH, 128), lambda i: (0, 0)),
            pl.BlockSpec((_BATCH, 128), lambda i: (0, 0)),
        ],
        out_shape=[
            jax.ShapeDtypeStruct((_BATCH, 128), jnp.float32),
            jax.ShapeDtypeStruct((_BATCH, 128), jnp.float32),
        ],
        scratch_shapes=[pltpu.VMEM((_BATCH, 1), jnp.int32)],
    )(inputs, featT, labrows, idx2d)


def _tc_finish_body(inp_ref, feat_ref, lab_ref, idx_ref, acc_ref, pick_ref,
                    out_ref):
    t = _extract_t(lab_ref[...], idx_ref[...])

    inp = (inp_ref[...] * _INV_TEMP).astype(jnp.bfloat16)
    logits = lax.dot_general(
        inp, feat_ref[...].astype(jnp.bfloat16),
        dimension_numbers=(((1,), (0,)), ((), ())),
        preferred_element_type=jnp.float32,
    )
    col = _TIDX * _TBLK + lax.broadcasted_iota(jnp.int32, (_BATCH, _TBLK), 1)
    e = jnp.where(col < _N, jnp.exp(logits - _SHIFT), 0.0)
    se = jnp.sum(acc_ref[...] + _lane_chunk_sum(e, _TBLK), axis=1,
                 keepdims=True)
    lse = jnp.log(se) + _SHIFT

    pk = pick_ref[...] + _pick_chunks(logits, t - _TIDX * _TBLK, _TBLK)
    lane = lax.broadcasted_iota(jnp.int32, (_BATCH, 128), 1)
    picked = jnp.sum(jnp.where(lane == (t & 127), pk, 0.0), axis=1,
                     keepdims=True)
    out_ref[...] = jnp.mean(lse - picked)[None, None]


def _tc_finish(inputs, featT, labrows, idx2d, acc, pick):
    return pl.pallas_call(
        _tc_finish_body,
        grid=(1,),
        in_specs=[
            pl.BlockSpec((_BATCH, _F), lambda i: (0, 0)),
            pl.BlockSpec((_F, _TBLK), lambda i: (0, _TIDX)),
            pl.BlockSpec((_BATCH, 128), lambda i: (0, 0)),
            pl.BlockSpec((_BATCH, 1), lambda i: (0, 0)),
            pl.BlockSpec((_BATCH, 128), lambda i: (0, 0)),
            pl.BlockSpec((_BATCH, 128), lambda i: (0, 0)),
        ],
        out_specs=pl.BlockSpec((1, 1), lambda i: (0, 0)),
        out_shape=jax.ShapeDtypeStruct((1, 1), jnp.float32),
    )(inputs, featT, labrows, idx2d, acc, pick)


def kernel(inputs, indices, features, labels):
    labpad = jnp.pad(labels, (0, 782 * 128 - _N)).reshape(782, 128)
    featT = features.T
    idx2d = indices.reshape(_BATCH, 1)
    labrows = _sc_gather128(indices, labpad, 7, jnp.int32)
    acc, pick = _tc_main(inputs, featT, labrows, idx2d)
    loss = _tc_finish(inputs, featT, labrows, idx2d, acc, pick)
    return loss[0, 0]
